# baseline (speedup 1.0000x reference)
import jax
import jax.numpy as jnp
from jax.experimental import pallas as pl

_B = 8
_N = 2048
_S = 512
_K = 32
_R2 = 0.4 * 0.4
_TOT = _B * _S * _K
_CH = 8
_NCHUNK = _S // _CH


def _fps_kernel(xyz_ref, ox_ref, oy_ref, oz_ref):
    x = xyz_ref[:, 0, :]
    y = xyz_ref[:, 1, :]
    z = xyz_ref[:, 2, :]
    iota = jax.lax.broadcasted_iota(jnp.int32, (_B, _N), 1)
    siota = jax.lax.broadcasted_iota(jnp.int32, (_B, _S), 1)

    def body(i, carry):
        dist, far, ax, ay, az = carry
        sel = iota == far
        cx = jnp.sum(jnp.where(sel, x, 0.0), axis=1, keepdims=True)
        cy = jnp.sum(jnp.where(sel, y, 0.0), axis=1, keepdims=True)
        cz = jnp.sum(jnp.where(sel, z, 0.0), axis=1, keepdims=True)
        col = siota == i
        ax = jnp.where(col, cx, ax)
        ay = jnp.where(col, cy, ay)
        az = jnp.where(col, cz, az)
        d = (x - cx) ** 2 + (y - cy) ** 2 + (z - cz) ** 2
        dist = jnp.minimum(dist, d)
        m = jnp.max(dist, axis=1, keepdims=True)
        far = jnp.min(jnp.where(dist == m, iota, _N), axis=1, keepdims=True)
        return dist, far, ax, ay, az

    zs = jnp.zeros((_B, _S), jnp.float32)
    init = (jnp.full((_B, _N), 1e10, dtype=jnp.float32),
            jnp.zeros((_B, 1), dtype=jnp.int32), zs, zs, zs)
    _, _, ax, ay, az = jax.lax.fori_loop(0, _S, body, init)
    ox_ref[...] = ax
    oy_ref[...] = ay
    oz_ref[...] = az


def _group_conv1_kernel(v_ref, c_ref, w_ref, b_ref, z_ref, s_ref, q_ref):
    b = pl.program_id(0)
    V = v_ref[0]
    W = w_ref[...]
    VW = jax.lax.dot_general(V, W, (((0,), (1,)), ((), ())),
                             preferred_element_type=jnp.float32)
    VWh = VW.astype(jnp.bfloat16)
    VWl = (VW - VWh.astype(jnp.float32)).astype(jnp.bfloat16)
    xr = V[0:1, :]
    yr = V[1:2, :]
    zr = V[2:3, :]

    def chunk(j, carry):
        ssum, ssq = carry
        Cc = c_ref[0, pl.ds(j * _CH, _CH), :]
        cx = Cc[:, 0:1]
        cy = Cc[:, 1:2]
        cz = Cc[:, 2:3]
        d = (cx - xr) ** 2 + (cy - yr) ** 2 + (cz - zr) ** 2
        maskf = jnp.where(d <= _R2, 1.0, 0.0)
        rank = maskf
        sh = 1
        while sh < _N:
            z = jnp.zeros((_CH, sh), jnp.float32)
            rank = rank + jnp.concatenate([z, rank[:, :_N - sh]], axis=1)
            sh *= 2
        cnt = rank[:, _N - 1:_N]
        rmask = rank * maskf
        rank3 = jnp.broadcast_to(rmask[:, None, :], (_CH, _K, _N)).reshape(_CH * _K, _N)
        kk = jax.lax.broadcasted_iota(jnp.int32, (_CH, _K, 1), 1).reshape(
            _CH * _K, 1).astype(jnp.float32)
        A = jnp.where(rank3 == (kk + 1.0), 1.0, 0.0).astype(jnp.bfloat16)
        Zc = (jnp.dot(A, VWh, preferred_element_type=jnp.float32)
              + jnp.dot(A, VWl, preferred_element_type=jnp.float32))
        Zc3 = Zc.reshape(_CH, _K, 64)
        first = Zc3[:, 0:1, :]
        km = jax.lax.broadcasted_iota(jnp.int32, (_CH, _K, 1), 1).astype(jnp.float32)
        Zc = jnp.where(km < cnt[:, :, None], Zc3, first).reshape(_CH * _K, 64)
        Corrc = jax.lax.dot_general(Cc, W[:, 0:3], (((1,), (1,)), ((), ())),
                                    preferred_element_type=jnp.float32)
        Corr3 = jnp.broadcast_to(Corrc[:, None, :], (_CH, _K, 64)).reshape(_CH * _K, 64)
        Zc = Zc - Corr3 + b_ref[...]
        z_ref[0, pl.ds(j * _CH * _K, _CH * _K), :] = Zc
        ssum = ssum + jnp.sum(Zc, axis=0, keepdims=True)
        ssq = ssq + jnp.sum(Zc * Zc, axis=0, keepdims=True)
        return ssum, ssq

    ssum, ssq = jax.lax.fori_loop(
        0, _NCHUNK, chunk,
        (jnp.zeros((1, 64), jnp.float32), jnp.zeros((1, 64), jnp.float32)))

    @pl.when(b == 0)
    def _():
        s_ref[...] = ssum
        q_ref[...] = ssq

    @pl.when(b != 0)
    def _():
        s_ref[...] = s_ref[...] + ssum
        q_ref[...] = q_ref[...] + ssq


def _bn_conv_kernel(z_ref, s_ref, q_ref, g_ref, be_ref, w_ref, bias_ref,
                    o_ref, so_ref, qo_ref):
    b = pl.program_id(0)
    j = pl.program_id(1)
    Z = z_ref[0]
    mu = s_ref[...] / _TOT
    var = q_ref[...] / _TOT - mu * mu
    inv = jax.lax.rsqrt(var + 1e-5)
    X = jnp.maximum(g_ref[...] * (Z - mu) * inv + be_ref[...], 0.0)
    Zo = jax.lax.dot_general(X, w_ref[...], (((1,), (1,)), ((), ())),
                             preferred_element_type=jnp.float32)
    Zo = Zo + bias_ref[...]
    o_ref[0] = Zo
    ssum = jnp.sum(Zo, axis=0, keepdims=True)
    ssq = jnp.sum(Zo * Zo, axis=0, keepdims=True)

    @pl.when((b == 0) & (j == 0))
    def _():
        so_ref[...] = ssum
        qo_ref[...] = ssq

    @pl.when((b != 0) | (j != 0))
    def _():
        so_ref[...] = so_ref[...] + ssum
        qo_ref[...] = qo_ref[...] + ssq


def _bn_max_kernel(z_ref, s_ref, q_ref, g_ref, be_ref, o_ref):
    Z = z_ref[0]
    mu = s_ref[...] / _TOT
    var = q_ref[...] / _TOT - mu * mu
    inv = jax.lax.rsqrt(var + 1e-5)
    X = jnp.maximum(g_ref[...] * (Z - mu) * inv + be_ref[...], 0.0)
    rows = Z.shape[0]
    Xr = X.reshape(rows // _K, _K, 128)
    o_ref[0] = jnp.max(Xr, axis=1)


def kernel(xyz, points, conv_w, conv_b, bn_g, bn_b):
    f32 = jnp.float32
    V = jnp.concatenate([xyz, points], axis=1)

    nx, ny, nz = pl.pallas_call(
        _fps_kernel,
        out_shape=[jax.ShapeDtypeStruct((_B, _S), f32)] * 3,
    )(xyz)
    new_xyz = jnp.stack([nx, ny, nz], axis=1)
    C = jnp.transpose(new_xyz, (0, 2, 1))

    W1, W2, W3 = conv_w
    b1 = conv_b[0].reshape(1, 64)
    b2 = conv_b[1].reshape(1, 64)
    b3 = conv_b[2].reshape(1, 128)
    g1 = bn_g[0].reshape(1, 64)
    g2 = bn_g[1].reshape(1, 64)
    g3 = bn_g[2].reshape(1, 128)
    e1 = bn_b[0].reshape(1, 64)
    e2 = bn_b[1].reshape(1, 64)
    e3 = bn_b[2].reshape(1, 128)

    SK = _S * _K
    Z1, s1, q1 = pl.pallas_call(
        _group_conv1_kernel,
        grid=(_B,),
        in_specs=[
            pl.BlockSpec((1, 67, _N), lambda b: (b, 0, 0)),
            pl.BlockSpec((1, _S, 3), lambda b: (b, 0, 0)),
            pl.BlockSpec((64, 67), lambda b: (0, 0)),
            pl.BlockSpec((1, 64), lambda b: (0, 0)),
        ],
        out_specs=[
            pl.BlockSpec((1, SK, 64), lambda b: (b, 0, 0)),
            pl.BlockSpec((1, 64), lambda b: (0, 0)),
            pl.BlockSpec((1, 64), lambda b: (0, 0)),
        ],
        out_shape=[
            jax.ShapeDtypeStruct((_B, SK, 64), f32),
            jax.ShapeDtypeStruct((1, 64), f32),
            jax.ShapeDtypeStruct((1, 64), f32),
        ],
    )(V, C, W1, b1)

    NJ = 8
    ROWS = SK // NJ

    def bn_conv(Z, s, q, g, be, W, bias, cout):
        cin = Z.shape[-1]
        return pl.pallas_call(
            _bn_conv_kernel,
            grid=(_B, NJ),
            in_specs=[
                pl.BlockSpec((1, ROWS, cin), lambda b, j: (b, j, 0)),
                pl.BlockSpec((1, cin), lambda b, j: (0, 0)),
                pl.BlockSpec((1, cin), lambda b, j: (0, 0)),
                pl.BlockSpec((1, cin), lambda b, j: (0, 0)),
                pl.BlockSpec((1, cin), lambda b, j: (0, 0)),
                pl.BlockSpec((cout, cin), lambda b, j: (0, 0)),
                pl.BlockSpec((1, cout), lambda b, j: (0, 0)),
            ],
            out_specs=[
                pl.BlockSpec((1, ROWS, cout), lambda b, j: (b, j, 0)),
                pl.BlockSpec((1, cout), lambda b, j: (0, 0)),
                pl.BlockSpec((1, cout), lambda b, j: (0, 0)),
            ],
            out_shape=[
                jax.ShapeDtypeStruct((_B, SK, cout), f32),
                jax.ShapeDtypeStruct((1, cout), f32),
                jax.ShapeDtypeStruct((1, cout), f32),
            ],
        )(Z, s, q, g, be, W, bias)

    Z2, s2, q2 = bn_conv(Z1, s1, q1, g1, e1, W2, b2, 64)
    Z3, s3, q3 = bn_conv(Z2, s2, q2, g2, e2, W3, b3, 128)

    P = pl.pallas_call(
        _bn_max_kernel,
        grid=(_B, NJ),
        in_specs=[
            pl.BlockSpec((1, ROWS, 128), lambda b, j: (b, j, 0)),
            pl.BlockSpec((1, 128), lambda b, j: (0, 0)),
            pl.BlockSpec((1, 128), lambda b, j: (0, 0)),
            pl.BlockSpec((1, 128), lambda b, j: (0, 0)),
            pl.BlockSpec((1, 128), lambda b, j: (0, 0)),
        ],
        out_specs=pl.BlockSpec((1, ROWS // _K, 128), lambda b, j: (b, j, 0)),
        out_shape=jax.ShapeDtypeStruct((_B, _S, 128), f32),
    )(Z3, s3, q3, g3, e3)

    new_points = jnp.transpose(P, (0, 2, 1))
    return (new_xyz, new_points)


# speedup vs baseline: 1.4721x; 1.4721x over previous
import jax
import jax.numpy as jnp
from jax.experimental import pallas as pl

_B = 8
_N = 2048
_S = 512
_K = 32
_R2 = 0.4 * 0.4
_TOT = _B * _S * _K
_CH = 16
_NCHUNK = _S // _CH


def _fps_kernel(xyz_ref, ox_ref, oy_ref, oz_ref):
    x = xyz_ref[:, 0, :]
    y = xyz_ref[:, 1, :]
    z = xyz_ref[:, 2, :]
    iota = jax.lax.broadcasted_iota(jnp.int32, (_B, _N), 1)
    siota = jax.lax.broadcasted_iota(jnp.int32, (_B, _S), 1)

    def body(i, carry):
        dist, far, ax, ay, az = carry
        sel = iota == far
        cx = jnp.sum(jnp.where(sel, x, 0.0), axis=1, keepdims=True)
        cy = jnp.sum(jnp.where(sel, y, 0.0), axis=1, keepdims=True)
        cz = jnp.sum(jnp.where(sel, z, 0.0), axis=1, keepdims=True)
        col = siota == i
        ax = jnp.where(col, cx, ax)
        ay = jnp.where(col, cy, ay)
        az = jnp.where(col, cz, az)
        d = (x - cx) ** 2 + (y - cy) ** 2 + (z - cz) ** 2
        dist = jnp.minimum(dist, d)
        m = jnp.max(dist, axis=1, keepdims=True)
        far = jnp.min(jnp.where(dist == m, iota, _N), axis=1, keepdims=True)
        return dist, far, ax, ay, az

    zs = jnp.zeros((_B, _S), jnp.float32)
    init = (jnp.full((_B, _N), 1e10, dtype=jnp.float32),
            jnp.zeros((_B, 1), dtype=jnp.int32), zs, zs, zs)
    _, _, ax, ay, az = jax.lax.fori_loop(0, _S, body, init)
    ox_ref[...] = ax
    oy_ref[...] = ay
    oz_ref[...] = az


def _group_conv1_kernel(v_ref, c_ref, w_ref, b_ref, z_ref, s_ref, q_ref):
    b = pl.program_id(0)
    V = v_ref[0]
    W = w_ref[...]
    VW = jax.lax.dot_general(V, W, (((0,), (1,)), ((), ())),
                             preferred_element_type=jnp.float32)
    xr = V[0:1, :]
    yr = V[1:2, :]
    zr = V[2:3, :]

    def chunk(j, carry):
        ssum, ssq = carry
        Cc = c_ref[0, pl.ds(j * _CH, _CH), :]
        cx = Cc[:, 0:1]
        cy = Cc[:, 1:2]
        cz = Cc[:, 2:3]
        d = (cx - xr) ** 2 + (cy - yr) ** 2 + (cz - zr) ** 2
        maskf = jnp.where(d <= _R2, 1.0, 0.0)
        rank = maskf
        sh = 1
        while sh < _N:
            z = jnp.zeros((_CH, sh), jnp.float32)
            rank = rank + jnp.concatenate([z, rank[:, :_N - sh]], axis=1)
            sh *= 2
        cnt = rank[:, _N - 1:_N]
        rmask = rank * maskf
        rank3 = jnp.broadcast_to(rmask[:, None, :], (_CH, _K, _N)).reshape(_CH * _K, _N)
        kk = jax.lax.broadcasted_iota(jnp.int32, (_CH, _K, 1), 1).reshape(
            _CH * _K, 1).astype(jnp.float32)
        A = jnp.where(rank3 == (kk + 1.0), 1.0, 0.0)
        Zc = jnp.dot(A, VW, preferred_element_type=jnp.float32)
        Zc3 = Zc.reshape(_CH, _K, 64)
        first = Zc3[:, 0:1, :]
        km = jax.lax.broadcasted_iota(jnp.int32, (_CH, _K, 1), 1).astype(jnp.float32)
        Zc = jnp.where(km < cnt[:, :, None], Zc3, first).reshape(_CH * _K, 64)
        Corrc = jax.lax.dot_general(Cc, W[:, 0:3], (((1,), (1,)), ((), ())),
                                    preferred_element_type=jnp.float32)
        Corr3 = jnp.broadcast_to(Corrc[:, None, :], (_CH, _K, 64)).reshape(_CH * _K, 64)
        Zc = Zc - Corr3 + b_ref[...]
        z_ref[0, pl.ds(j * _CH * _K, _CH * _K), :] = Zc
        ssum = ssum + jnp.sum(Zc, axis=0, keepdims=True)
        ssq = ssq + jnp.sum(Zc * Zc, axis=0, keepdims=True)
        return ssum, ssq

    ssum, ssq = jax.lax.fori_loop(
        0, _NCHUNK, chunk,
        (jnp.zeros((1, 64), jnp.float32), jnp.zeros((1, 64), jnp.float32)))

    @pl.when(b == 0)
    def _():
        s_ref[...] = ssum
        q_ref[...] = ssq

    @pl.when(b != 0)
    def _():
        s_ref[...] = s_ref[...] + ssum
        q_ref[...] = q_ref[...] + ssq


def _bn_conv_kernel(z_ref, s_ref, q_ref, g_ref, be_ref, w_ref, bias_ref,
                    o_ref, so_ref, qo_ref):
    b = pl.program_id(0)
    j = pl.program_id(1)
    Z = z_ref[0]
    mu = s_ref[...] / _TOT
    var = q_ref[...] / _TOT - mu * mu
    inv = jax.lax.rsqrt(var + 1e-5)
    X = jnp.maximum(g_ref[...] * (Z - mu) * inv + be_ref[...], 0.0)
    Zo = jax.lax.dot_general(X, w_ref[...], (((1,), (1,)), ((), ())),
                             preferred_element_type=jnp.float32)
    Zo = Zo + bias_ref[...]
    o_ref[0] = Zo
    ssum = jnp.sum(Zo, axis=0, keepdims=True)
    ssq = jnp.sum(Zo * Zo, axis=0, keepdims=True)

    @pl.when((b == 0) & (j == 0))
    def _():
        so_ref[...] = ssum
        qo_ref[...] = ssq

    @pl.when((b != 0) | (j != 0))
    def _():
        so_ref[...] = so_ref[...] + ssum
        qo_ref[...] = qo_ref[...] + ssq


def _bn_max_kernel(z_ref, s_ref, q_ref, g_ref, be_ref, o_ref):
    Z = z_ref[0]
    mu = s_ref[...] / _TOT
    var = q_ref[...] / _TOT - mu * mu
    inv = jax.lax.rsqrt(var + 1e-5)
    X = jnp.maximum(g_ref[...] * (Z - mu) * inv + be_ref[...], 0.0)
    rows = Z.shape[0]
    Xr = X.reshape(rows // _K, _K, 128)
    o_ref[0] = jnp.max(Xr, axis=1)


def kernel(xyz, points, conv_w, conv_b, bn_g, bn_b):
    f32 = jnp.float32
    V = jnp.concatenate([xyz, points], axis=1)

    nx, ny, nz = pl.pallas_call(
        _fps_kernel,
        out_shape=[jax.ShapeDtypeStruct((_B, _S), f32)] * 3,
    )(xyz)
    new_xyz = jnp.stack([nx, ny, nz], axis=1)
    C = jnp.transpose(new_xyz, (0, 2, 1))

    W1, W2, W3 = conv_w
    b1 = conv_b[0].reshape(1, 64)
    b2 = conv_b[1].reshape(1, 64)
    b3 = conv_b[2].reshape(1, 128)
    g1 = bn_g[0].reshape(1, 64)
    g2 = bn_g[1].reshape(1, 64)
    g3 = bn_g[2].reshape(1, 128)
    e1 = bn_b[0].reshape(1, 64)
    e2 = bn_b[1].reshape(1, 64)
    e3 = bn_b[2].reshape(1, 128)

    SK = _S * _K
    Z1, s1, q1 = pl.pallas_call(
        _group_conv1_kernel,
        grid=(_B,),
        in_specs=[
            pl.BlockSpec((1, 67, _N), lambda b: (b, 0, 0)),
            pl.BlockSpec((1, _S, 3), lambda b: (b, 0, 0)),
            pl.BlockSpec((64, 67), lambda b: (0, 0)),
            pl.BlockSpec((1, 64), lambda b: (0, 0)),
        ],
        out_specs=[
            pl.BlockSpec((1, SK, 64), lambda b: (b, 0, 0)),
            pl.BlockSpec((1, 64), lambda b: (0, 0)),
            pl.BlockSpec((1, 64), lambda b: (0, 0)),
        ],
        out_shape=[
            jax.ShapeDtypeStruct((_B, SK, 64), f32),
            jax.ShapeDtypeStruct((1, 64), f32),
            jax.ShapeDtypeStruct((1, 64), f32),
        ],
    )(V, C, W1, b1)

    NJ = 8
    ROWS = SK // NJ

    def bn_conv(Z, s, q, g, be, W, bias, cout):
        cin = Z.shape[-1]
        return pl.pallas_call(
            _bn_conv_kernel,
            grid=(_B, NJ),
            in_specs=[
                pl.BlockSpec((1, ROWS, cin), lambda b, j: (b, j, 0)),
                pl.BlockSpec((1, cin), lambda b, j: (0, 0)),
                pl.BlockSpec((1, cin), lambda b, j: (0, 0)),
                pl.BlockSpec((1, cin), lambda b, j: (0, 0)),
                pl.BlockSpec((1, cin), lambda b, j: (0, 0)),
                pl.BlockSpec((cout, cin), lambda b, j: (0, 0)),
                pl.BlockSpec((1, cout), lambda b, j: (0, 0)),
            ],
            out_specs=[
                pl.BlockSpec((1, ROWS, cout), lambda b, j: (b, j, 0)),
                pl.BlockSpec((1, cout), lambda b, j: (0, 0)),
                pl.BlockSpec((1, cout), lambda b, j: (0, 0)),
            ],
            out_shape=[
                jax.ShapeDtypeStruct((_B, SK, cout), f32),
                jax.ShapeDtypeStruct((1, cout), f32),
                jax.ShapeDtypeStruct((1, cout), f32),
            ],
        )(Z, s, q, g, be, W, bias)

    Z2, s2, q2 = bn_conv(Z1, s1, q1, g1, e1, W2, b2, 64)
    Z3, s3, q3 = bn_conv(Z2, s2, q2, g2, e2, W3, b3, 128)

    P = pl.pallas_call(
        _bn_max_kernel,
        grid=(_B, NJ),
        in_specs=[
            pl.BlockSpec((1, ROWS, 128), lambda b, j: (b, j, 0)),
            pl.BlockSpec((1, 128), lambda b, j: (0, 0)),
            pl.BlockSpec((1, 128), lambda b, j: (0, 0)),
            pl.BlockSpec((1, 128), lambda b, j: (0, 0)),
            pl.BlockSpec((1, 128), lambda b, j: (0, 0)),
        ],
        out_specs=pl.BlockSpec((1, ROWS // _K, 128), lambda b, j: (b, j, 0)),
        out_shape=jax.ShapeDtypeStruct((_B, _S, 128), f32),
    )(Z3, s3, q3, g3, e3)

    new_points = jnp.transpose(P, (0, 2, 1))
    return (new_xyz, new_points)
